# baseline (speedup 1.0000x reference)
import jax
import jax.numpy as jnp
from jax.experimental import pallas as pl

T, D, FF, E, R = 2048, 1024, 2816, 8, 16
ER = E * R
TOP_K = 2
LORA_SCALE = 2.0
TB = 256


def _mask_kernel(gv_ref, me_ref):
    f32 = jnp.float32
    gv = gv_ref[...]
    vj = gv[:, None, :]
    ve = gv[:, :, None]
    j_idx = jax.lax.broadcasted_iota(jnp.int32, (T, E, E), 2)
    e_idx = jax.lax.broadcasted_iota(jnp.int32, (T, E, E), 1)
    beats = jnp.logical_or(vj > ve, jnp.logical_and(vj == ve, j_idx < e_idx))
    rank = jnp.sum(beats.astype(jnp.int32), axis=2)
    mask = jnp.where(rank < TOP_K, LORA_SCALE, 0.0).astype(f32)
    sel_r = jax.lax.broadcasted_iota(jnp.int32, (E, ER), 0)
    sel_c = jax.lax.broadcasted_iota(jnp.int32, (E, ER), 1)
    sel = (sel_r == sel_c // R).astype(f32)
    me_ref[...] = jnp.dot(mask, sel, preferred_element_type=f32)


def _mlp_kernel(me_ref, x_ref, wg_ref, wu_ref, wd_ref, agu_ref, bg_ref,
                bu_ref, ad_ref, bd_ref, out_ref):
    f32 = jnp.float32
    bf16 = jnp.bfloat16
    me = me_ref[...]
    xb = x_ref[...]
    mid = jnp.dot(xb, agu_ref[...], preferred_element_type=f32)
    mid_g = (mid[:, :ER] * me).astype(bf16)
    mid_u = (mid[:, ER:] * me).astype(bf16)
    g = (jnp.dot(xb, wg_ref[...], preferred_element_type=f32)
         + jnp.dot(mid_g, bg_ref[...], preferred_element_type=f32))
    u = (jnp.dot(xb, wu_ref[...], preferred_element_type=f32)
         + jnp.dot(mid_u, bu_ref[...], preferred_element_type=f32))
    gb = g.astype(bf16)
    hb = gb * jax.nn.sigmoid(gb) * u.astype(bf16)
    mid_d = (jnp.dot(hb, ad_ref[...], preferred_element_type=f32) * me
             ).astype(bf16)
    out_ref[...] = (
        jnp.dot(hb, wd_ref[...], preferred_element_type=f32)
        + jnp.dot(mid_d, bd_ref[...], preferred_element_type=f32))


@jax.jit
def kernel(x, gate_values, W_gate, W_up, W_down, A_gate, B_gate, A_up, B_up,
           A_down, B_down):
    bf16 = jnp.bfloat16
    xb = x.astype(bf16)
    ag = A_gate.transpose(1, 0, 2).reshape(D, ER)
    au = A_up.transpose(1, 0, 2).reshape(D, ER)
    agu = jnp.concatenate([ag, au], axis=1).astype(bf16)
    ad = A_down.transpose(1, 0, 2).reshape(FF, ER).astype(bf16)
    bg = B_gate.reshape(ER, FF).astype(bf16)
    bu = B_up.reshape(ER, FF).astype(bf16)
    bd = B_down.reshape(ER, D).astype(bf16)

    me = pl.pallas_call(
        _mask_kernel,
        out_shape=jax.ShapeDtypeStruct((T, ER), jnp.float32),
    )(gate_values)

    grid = (T // TB,)
    tok = lambda i: (i, 0)
    full = lambda i: (0, 0)
    out = pl.pallas_call(
        _mlp_kernel,
        grid=grid,
        in_specs=[
            pl.BlockSpec((TB, ER), tok),
            pl.BlockSpec((TB, D), tok),
            pl.BlockSpec((D, FF), full),
            pl.BlockSpec((D, FF), full),
            pl.BlockSpec((FF, D), full),
            pl.BlockSpec((D, 2 * ER), full),
            pl.BlockSpec((ER, FF), full),
            pl.BlockSpec((ER, FF), full),
            pl.BlockSpec((FF, ER), full),
            pl.BlockSpec((ER, D), full),
        ],
        out_specs=pl.BlockSpec((TB, D), tok),
        out_shape=jax.ShapeDtypeStruct((T, D), jnp.float32),
    )(me, xb, W_gate.astype(bf16), W_up.astype(bf16),
      W_down.astype(bf16), agu, bg, bu, ad, bd)
    return out


# speedup vs baseline: 1.0166x; 1.0166x over previous
import jax
import jax.numpy as jnp
from jax.experimental import pallas as pl

T, D, FF, E, R = 2048, 1024, 2816, 8, 16
ER = E * R
TOP_K = 2
LORA_SCALE = 2.0
TB = 256


def _mlp_kernel(gv_ref, x_ref, wg_ref, wu_ref, wd_ref, ag_ref, bg_ref,
                au_ref, bu_ref, ad_ref, bd_ref, out_ref):
    f32 = jnp.float32
    gv = gv_ref[...]
    vj = gv[:, None, :]
    ve = gv[:, :, None]
    j_idx = jax.lax.broadcasted_iota(jnp.int32, (TB, E, E), 2)
    e_idx = jax.lax.broadcasted_iota(jnp.int32, (TB, E, E), 1)
    beats = jnp.logical_or(vj > ve, jnp.logical_and(vj == ve, j_idx < e_idx))
    rank = jnp.sum(beats.astype(jnp.int32), axis=2)
    mask = jnp.where(rank < TOP_K, LORA_SCALE, 0.0).astype(f32)
    sel_r = jax.lax.broadcasted_iota(jnp.int32, (E, ER), 0)
    sel_c = jax.lax.broadcasted_iota(jnp.int32, (E, ER), 1)
    sel = (sel_r == sel_c // R).astype(f32)
    me = jnp.dot(mask, sel, preferred_element_type=f32)

    xb = x_ref[...]
    mid_g = jnp.dot(xb, ag_ref[...], preferred_element_type=f32)
    mid_u = jnp.dot(xb, au_ref[...], preferred_element_type=f32)
    mid_g = (mid_g * me).astype(jnp.bfloat16)
    mid_u = (mid_u * me).astype(jnp.bfloat16)
    g = (jnp.dot(xb, wg_ref[...], preferred_element_type=f32)
         + jnp.dot(mid_g, bg_ref[...], preferred_element_type=f32))
    u = (jnp.dot(xb, wu_ref[...], preferred_element_type=f32)
         + jnp.dot(mid_u, bu_ref[...], preferred_element_type=f32))
    h = (g * jax.nn.sigmoid(g)) * u
    hb = h.astype(jnp.bfloat16)
    mid_d = jnp.dot(hb, ad_ref[...], preferred_element_type=f32)
    mid_d = (mid_d * me).astype(jnp.bfloat16)
    out_ref[...] = (
        jnp.dot(hb, wd_ref[...], preferred_element_type=f32)
        + jnp.dot(mid_d, bd_ref[...], preferred_element_type=f32))


@jax.jit
def kernel(x, gate_values, W_gate, W_up, W_down, A_gate, B_gate, A_up, B_up,
           A_down, B_down):
    bf16 = jnp.bfloat16
    xb = x.astype(bf16)
    ag = A_gate.transpose(1, 0, 2).reshape(D, ER).astype(bf16)
    au = A_up.transpose(1, 0, 2).reshape(D, ER).astype(bf16)
    ad = A_down.transpose(1, 0, 2).reshape(FF, ER).astype(bf16)
    bg = B_gate.reshape(ER, FF).astype(bf16)
    bu = B_up.reshape(ER, FF).astype(bf16)
    bd = B_down.reshape(ER, D).astype(bf16)

    grid = (T // TB,)
    tok = lambda i: (i, 0)
    full = lambda i: (0, 0)
    out = pl.pallas_call(
        _mlp_kernel,
        grid=grid,
        in_specs=[
            pl.BlockSpec((TB, E), tok),
            pl.BlockSpec((TB, D), tok),
            pl.BlockSpec((D, FF), full),
            pl.BlockSpec((D, FF), full),
            pl.BlockSpec((FF, D), full),
            pl.BlockSpec((D, ER), full),
            pl.BlockSpec((ER, FF), full),
            pl.BlockSpec((D, ER), full),
            pl.BlockSpec((ER, FF), full),
            pl.BlockSpec((FF, ER), full),
            pl.BlockSpec((ER, D), full),
        ],
        out_specs=pl.BlockSpec((TB, D), tok),
        out_shape=jax.ShapeDtypeStruct((T, D), jnp.float32),
    )(gate_values, xb, W_gate.astype(bf16), W_up.astype(bf16),
      W_down.astype(bf16), ag, bg, au, bu, ad, bd)
    return out


# speedup vs baseline: 1.6501x; 1.6231x over previous
import jax
import jax.numpy as jnp
from jax.experimental import pallas as pl

T, D, FF, E, R = 2048, 1024, 2816, 8, 16
ER = E * R
TOP_K = 2
LORA_SCALE = 2.0
TB = 256


def _mlp_kernel(gv_ref, x_ref, wg_ref, wu_ref, wd_ref, ag_ref, bg_ref,
                au_ref, bu_ref, ad_ref, bd_ref, out_ref):
    f32 = jnp.float32
    bf16 = jnp.bfloat16
    gv = gv_ref[...]
    vj = gv[:, None, :]
    ve = gv[:, :, None]
    j_idx = jax.lax.broadcasted_iota(jnp.int32, (TB, E, E), 2)
    e_idx = jax.lax.broadcasted_iota(jnp.int32, (TB, E, E), 1)
    beats = jnp.logical_or(vj > ve, jnp.logical_and(vj == ve, j_idx < e_idx))
    rank = jnp.sum(beats.astype(jnp.int32), axis=2)
    mask = (rank < TOP_K).astype(f32)
    sel_r = jax.lax.broadcasted_iota(jnp.int32, (E, ER), 0)
    sel_c = jax.lax.broadcasted_iota(jnp.int32, (E, ER), 1)
    sel = (sel_r == sel_c // R).astype(f32)
    me = jnp.dot(mask, sel, preferred_element_type=f32)

    xb = x_ref[...].astype(bf16)
    mid_g = jnp.dot(xb, ag_ref[...].astype(bf16), preferred_element_type=f32)
    mid_u = jnp.dot(xb, au_ref[...].astype(bf16), preferred_element_type=f32)
    mid_g = (mid_g * me).astype(bf16)
    mid_u = (mid_u * me).astype(bf16)
    g = (jnp.dot(xb, wg_ref[...].astype(bf16), preferred_element_type=f32)
         + LORA_SCALE * jnp.dot(mid_g, bg_ref[...].astype(bf16),
                                preferred_element_type=f32))
    u = (jnp.dot(xb, wu_ref[...].astype(bf16), preferred_element_type=f32)
         + LORA_SCALE * jnp.dot(mid_u, bu_ref[...].astype(bf16),
                                preferred_element_type=f32))
    h = (g * jax.nn.sigmoid(g)) * u
    hb = h.astype(bf16)
    mid_d = jnp.dot(hb, ad_ref[...].astype(bf16), preferred_element_type=f32)
    mid_d = (mid_d * me).astype(bf16)
    out_ref[...] = (
        jnp.dot(hb, wd_ref[...].astype(bf16), preferred_element_type=f32)
        + LORA_SCALE * jnp.dot(mid_d, bd_ref[...].astype(bf16),
                               preferred_element_type=f32))


@jax.jit
def kernel(x, gate_values, W_gate, W_up, W_down, A_gate, B_gate, A_up, B_up,
           A_down, B_down):
    ag = A_gate.transpose(1, 0, 2).reshape(D, ER)
    au = A_up.transpose(1, 0, 2).reshape(D, ER)
    ad = A_down.transpose(1, 0, 2).reshape(FF, ER)
    bg = B_gate.reshape(ER, FF)
    bu = B_up.reshape(ER, FF)
    bd = B_down.reshape(ER, D)

    grid = (T // TB,)
    tok = lambda i: (i, 0)
    full = lambda i: (0, 0)
    out = pl.pallas_call(
        _mlp_kernel,
        grid=grid,
        in_specs=[
            pl.BlockSpec((TB, E), tok),
            pl.BlockSpec((TB, D), tok),
            pl.BlockSpec((D, FF), full),
            pl.BlockSpec((D, FF), full),
            pl.BlockSpec((FF, D), full),
            pl.BlockSpec((D, ER), full),
            pl.BlockSpec((ER, FF), full),
            pl.BlockSpec((D, ER), full),
            pl.BlockSpec((ER, FF), full),
            pl.BlockSpec((FF, ER), full),
            pl.BlockSpec((ER, D), full),
        ],
        out_specs=pl.BlockSpec((TB, D), tok),
        out_shape=jax.ShapeDtypeStruct((T, D), jnp.float32),
    )(gate_values, x, W_gate, W_up, W_down, ag, bg, au, bu, ad, bd)
    return out


# speedup vs baseline: 1.8490x; 1.1205x over previous
import jax
import jax.numpy as jnp
from jax.experimental import pallas as pl

T, D, FF, E, R = 2048, 1024, 2816, 8, 16
ER = E * R
TOP_K = 2
LORA_SCALE = 2.0
TB = 256


def _mlp_kernel(gv_ref, x_ref, wg_ref, wu_ref, wd_ref, ag_ref, bg_ref,
                au_ref, bu_ref, ad_ref, bd_ref, out_ref):
    f32 = jnp.float32
    bf16 = jnp.bfloat16
    gvt = gv_ref[...]
    ik = jax.lax.bitcast_convert_type(gvt, jnp.int32)
    key = jnp.where(ik < 0, ik ^ jnp.int32(0x7FFFFFFF), ik)
    e_idx = jax.lax.broadcasted_iota(jnp.int32, (E, TB), 0)
    rank = jnp.zeros((E, TB), f32)
    for j in range(E):
        kj = jnp.broadcast_to(key[j:j + 1, :], (E, TB))
        ge = jnp.where(kj >= key, 1.0, 0.0)
        gt = jnp.where(kj > key, 1.0, 0.0)
        rank = rank + jnp.where(e_idx > j, ge, gt)
    mask = jnp.transpose((rank < TOP_K).astype(f32))
    sel_r = jax.lax.broadcasted_iota(jnp.int32, (E, ER), 0)
    sel_c = jax.lax.broadcasted_iota(jnp.int32, (E, ER), 1)
    sel = (sel_r == sel_c // R).astype(f32)
    me = jnp.dot(mask, sel, preferred_element_type=f32)

    xb = x_ref[...].astype(bf16)
    mid_g = jnp.dot(xb, ag_ref[...].astype(bf16), preferred_element_type=f32)
    mid_u = jnp.dot(xb, au_ref[...].astype(bf16), preferred_element_type=f32)
    mid_g = (mid_g * me).astype(bf16)
    mid_u = (mid_u * me).astype(bf16)
    g = (jnp.dot(xb, wg_ref[...].astype(bf16), preferred_element_type=f32)
         + LORA_SCALE * jnp.dot(mid_g, bg_ref[...].astype(bf16),
                                preferred_element_type=f32))
    u = (jnp.dot(xb, wu_ref[...].astype(bf16), preferred_element_type=f32)
         + LORA_SCALE * jnp.dot(mid_u, bu_ref[...].astype(bf16),
                                preferred_element_type=f32))
    h = (g * jax.nn.sigmoid(g)) * u
    hb = h.astype(bf16)
    mid_d = jnp.dot(hb, ad_ref[...].astype(bf16), preferred_element_type=f32)
    mid_d = (mid_d * me).astype(bf16)
    out_ref[...] = (
        jnp.dot(hb, wd_ref[...].astype(bf16), preferred_element_type=f32)
        + LORA_SCALE * jnp.dot(mid_d, bd_ref[...].astype(bf16),
                               preferred_element_type=f32))


@jax.jit
def kernel(x, gate_values, W_gate, W_up, W_down, A_gate, B_gate, A_up, B_up,
           A_down, B_down):
    ag = A_gate.transpose(1, 0, 2).reshape(D, ER)
    au = A_up.transpose(1, 0, 2).reshape(D, ER)
    ad = A_down.transpose(1, 0, 2).reshape(FF, ER)
    bg = B_gate.reshape(ER, FF)
    bu = B_up.reshape(ER, FF)
    bd = B_down.reshape(ER, D)

    gvt = gate_values.T

    grid = (T // TB,)
    tok = lambda i: (i, 0)
    tokc = lambda i: (0, i)
    full = lambda i: (0, 0)
    out = pl.pallas_call(
        _mlp_kernel,
        grid=grid,
        in_specs=[
            pl.BlockSpec((E, TB), tokc),
            pl.BlockSpec((TB, D), tok),
            pl.BlockSpec((D, FF), full),
            pl.BlockSpec((D, FF), full),
            pl.BlockSpec((FF, D), full),
            pl.BlockSpec((D, ER), full),
            pl.BlockSpec((ER, FF), full),
            pl.BlockSpec((D, ER), full),
            pl.BlockSpec((ER, FF), full),
            pl.BlockSpec((FF, ER), full),
            pl.BlockSpec((ER, D), full),
        ],
        out_specs=pl.BlockSpec((TB, D), tok),
        out_shape=jax.ShapeDtypeStruct((T, D), jnp.float32),
    )(gvt, x, W_gate, W_up, W_down, ag, bg, au, bu, ad, bd)
    return out


# speedup vs baseline: 1.8514x; 1.0013x over previous
import jax
import jax.numpy as jnp
from jax.experimental import pallas as pl
from jax.experimental.pallas import tpu as pltpu

T, D, FF, E, R = 2048, 1024, 2816, 8, 16
ER = E * R
TOP_K = 2
LORA_SCALE = 2.0
TB = 256


def _mlp_kernel(gv_ref, x_ref, wg_ref, wu_ref, wd_ref, ag_ref, bg_ref,
                au_ref, bu_ref, ad_ref, bd_ref, out_ref):
    f32 = jnp.float32
    bf16 = jnp.bfloat16
    gvt = gv_ref[...]
    ik = jax.lax.bitcast_convert_type(gvt, jnp.int32)
    key = jnp.where(ik < 0, ik ^ jnp.int32(0x7FFFFFFF), ik)
    e_idx = jax.lax.broadcasted_iota(jnp.int32, (E, TB), 0)
    rank = jnp.zeros((E, TB), f32)
    for j in range(E):
        kj = jnp.broadcast_to(key[j:j + 1, :], (E, TB))
        ge = jnp.where(kj >= key, 1.0, 0.0)
        gt = jnp.where(kj > key, 1.0, 0.0)
        rank = rank + jnp.where(e_idx > j, ge, gt)
    mask = jnp.transpose((rank < TOP_K).astype(f32))
    sel_r = jax.lax.broadcasted_iota(jnp.int32, (E, ER), 0)
    sel_c = jax.lax.broadcasted_iota(jnp.int32, (E, ER), 1)
    sel = (sel_r == sel_c // R).astype(f32)
    me = jnp.dot(mask, sel, preferred_element_type=f32)

    xb = x_ref[...].astype(bf16)
    mid_g = jnp.dot(xb, ag_ref[...].astype(bf16), preferred_element_type=f32)
    mid_u = jnp.dot(xb, au_ref[...].astype(bf16), preferred_element_type=f32)
    mid_g = (mid_g * me).astype(bf16)
    mid_u = (mid_u * me).astype(bf16)
    g = (jnp.dot(xb, wg_ref[...].astype(bf16), preferred_element_type=f32)
         + LORA_SCALE * jnp.dot(mid_g, bg_ref[...].astype(bf16),
                                preferred_element_type=f32))
    u = (jnp.dot(xb, wu_ref[...].astype(bf16), preferred_element_type=f32)
         + LORA_SCALE * jnp.dot(mid_u, bu_ref[...].astype(bf16),
                                preferred_element_type=f32))
    h = (g * jax.nn.sigmoid(g)) * u
    hb = h.astype(bf16)
    mid_d = jnp.dot(hb, ad_ref[...].astype(bf16), preferred_element_type=f32)
    mid_d = (mid_d * me).astype(bf16)
    out_ref[...] = (
        jnp.dot(hb, wd_ref[...].astype(bf16), preferred_element_type=f32)
        + LORA_SCALE * jnp.dot(mid_d, bd_ref[...].astype(bf16),
                               preferred_element_type=f32))


@jax.jit
def kernel(x, gate_values, W_gate, W_up, W_down, A_gate, B_gate, A_up, B_up,
           A_down, B_down):
    ag = A_gate.transpose(1, 0, 2).reshape(D, ER)
    au = A_up.transpose(1, 0, 2).reshape(D, ER)
    ad = A_down.transpose(1, 0, 2).reshape(FF, ER)
    bg = B_gate.reshape(ER, FF)
    bu = B_up.reshape(ER, FF)
    bd = B_down.reshape(ER, D)

    gvt = gate_values.T

    grid = (T // TB,)
    tok = lambda i: (i, 0)
    tokc = lambda i: (0, i)
    full = lambda i: (0, 0)
    out = pl.pallas_call(
        _mlp_kernel,
        grid=grid,
        in_specs=[
            pl.BlockSpec((E, TB), tokc),
            pl.BlockSpec((TB, D), tok),
            pl.BlockSpec((D, FF), full),
            pl.BlockSpec((D, FF), full),
            pl.BlockSpec((FF, D), full),
            pl.BlockSpec((D, ER), full),
            pl.BlockSpec((ER, FF), full),
            pl.BlockSpec((D, ER), full),
            pl.BlockSpec((ER, FF), full),
            pl.BlockSpec((FF, ER), full),
            pl.BlockSpec((ER, D), full),
        ],
        out_specs=pl.BlockSpec((TB, D), tok),
        out_shape=jax.ShapeDtypeStruct((T, D), jnp.float32),
        compiler_params=pltpu.CompilerParams(
            dimension_semantics=("parallel",)),
    )(gvt, x, W_gate, W_up, W_down, ag, bg, au, bu, ad, bd)
    return out
